# baseline (speedup 1.0000x reference)
import functools

import jax
import jax.numpy as jnp
from jax import lax
from jax.experimental import pallas as pl
from jax.experimental.pallas import tpu as pltpu
from jax.experimental.pallas import tpu_sc as plsc

_B = 1024
_L = 512
_M = 64
_LANES = 16

_NQ = 112
_NP = 32
_T = _NQ + _NP
_QROWS = 16 * _NQ
_ROWS = _QROWS + 4 * _NP

_info = plsc.get_sparse_core_info()
_NC = _info.num_cores
_NS = _info.num_subcores
_NW = _NC * _NS
_SPT = _B // _NW

_HI = jax.lax.Precision.HIGHEST


def _prep_body(x_ref, eps_ref, tab_ref, idx_ref):
    e0 = eps_ref[0]
    e1 = eps_ref[1]

    def selT(sel, e):
        return lax.dot_general(sel, e, (((0,), (1,)), ((), ())),
                               preferred_element_type=jnp.float32,
                               precision=_HI)

    li = lax.broadcasted_iota(jnp.int32, (_L, _NQ), 0)
    qi = lax.broadcasted_iota(jnp.int32, (_L, _NQ), 1)
    s = [(li == 4 * qi + j).astype(jnp.float32) for j in range(4)]
    eq = [[selT(s[j], e0), selT(s[j], e1)] for j in range(4)]
    for c in range(16):
        tc = (eq[0][(c >> 3) & 1] * eq[1][(c >> 2) & 1]
              * eq[2][(c >> 1) & 1] * eq[3][c & 1])
        tab_ref[pl.ds(c * _NQ, _NQ), :] = tc

    li2 = lax.broadcasted_iota(jnp.int32, (_L, _NP), 0)
    pi2 = lax.broadcasted_iota(jnp.int32, (_L, _NP), 1)
    pa = (li2 == 448 + 2 * pi2).astype(jnp.float32)
    pb = (li2 == 449 + 2 * pi2).astype(jnp.float32)
    ea = [selT(pa, e0), selT(pa, e1)]
    eb = [selT(pb, e0), selT(pb, e1)]
    for c in range(4):
        tab_ref[pl.ds(_QROWS + c * _NP, _NP), :] = ea[c >> 1] * eb[c & 1]

    wq = (s[0] * 8.0 + s[1] * 4.0 + s[2] * 2.0 + s[3]) * (_NQ * _M)
    wp = (pa * 2.0 + pb) * (_NP * _M)
    w = jnp.concatenate([wq, wp], axis=1)
    xf = x_ref[...].astype(jnp.float32)
    combo = lax.dot_general(xf, w, (((1,), (0,)), ((), ())),
                            preferred_element_type=jnp.float32,
                            precision=_HI)
    tcol = lax.broadcasted_iota(jnp.int32, (_B, _T), 1)
    off = tcol * _M + jnp.where(tcol >= _NQ, (_QROWS - _NQ) * _M, 0)
    idx_ref[...] = combo.astype(jnp.int32) + off


def _sc_body(idx_hbm, table_hbm, out_hbm, idx_v, tab_v, out_v):
    sid = lax.axis_index("s")
    wid = sid * _NC + lax.axis_index("c")
    base = wid * _SPT

    pltpu.sync_copy(table_hbm, tab_v)
    pltpu.sync_copy(idx_hbm.at[pl.ds(base * _T, _SPT * _T)], idx_v)

    lane = lax.iota(jnp.int32, 16)
    ones = jnp.ones((_LANES,), jnp.float32)

    for g in range(_SPT // _LANES):

        def sample_body(j, outvec, g=g):
            i = g * _LANES + j

            def group_body(q, accs):
                a0, a1, a2, a3 = accs
                rbv = idx_v[pl.ds(i * _T + q * _LANES, _LANES)]
                for k in range(_LANES):
                    rb = rbv[k]
                    a0 = a0 * tab_v[pl.ds(rb, 16)]
                    a1 = a1 * tab_v[pl.ds(rb + 16, 16)]
                    a2 = a2 * tab_v[pl.ds(rb + 32, 16)]
                    a3 = a3 * tab_v[pl.ds(rb + 48, 16)]
                return (a0, a1, a2, a3)

            a0, a1, a2, a3 = lax.fori_loop(
                0, _T // _LANES, group_body, (ones, ones, ones, ones),
                unroll=3)
            tot = jnp.sum((a0 + a1) + (a2 + a3), axis=0)
            return jnp.where(lane == j, tot, outvec)

        outvec = lax.fori_loop(0, _LANES, sample_body,
                               jnp.zeros((_LANES,), jnp.float32))
        out_v[pl.ds(g * _LANES, _LANES)] = outvec

    pltpu.sync_copy(out_v, out_hbm.at[pl.ds(base, _SPT)])


@jax.jit
def _qgps(x, epsilon):
    tab, idx = pl.pallas_call(
        _prep_body,
        out_shape=(
            jax.ShapeDtypeStruct((_ROWS, _M), jnp.float32),
            jax.ShapeDtypeStruct((_B, _T), jnp.int32),
        ),
    )(x, epsilon)

    run = pl.kernel(
        _sc_body,
        out_type=jax.ShapeDtypeStruct((_B,), jnp.float32),
        mesh=plsc.VectorSubcoreMesh(core_axis_name="c", subcore_axis_name="s"),
        scratch_types=[
            pltpu.VMEM((_SPT * _T,), jnp.int32),
            pltpu.VMEM((_ROWS * _M,), jnp.float32),
            pltpu.VMEM((_SPT,), jnp.float32),
        ],
        compiler_params=pltpu.CompilerParams(needs_layout_passes=False),
    )
    return run(idx.reshape(-1), tab.reshape(-1))


def kernel(inputs, epsilon):
    return _qgps(inputs.astype(jnp.int32), epsilon)


# speedup vs baseline: 1.3834x; 1.3834x over previous
import functools

import jax
import jax.numpy as jnp
from jax import lax
from jax.experimental import pallas as pl
from jax.experimental.pallas import tpu as pltpu
from jax.experimental.pallas import tpu_sc as plsc

_B = 1024
_L = 512
_M = 64
_LANES = 16

_NT = 160
_NP = 16
_T = _NT + _NP
_TROWS = 8 * _NT
_ROWS = _TROWS + 4 * _NP

_info = plsc.get_sparse_core_info()
_NC = _info.num_cores
_NS = _info.num_subcores
_NW = _NC * _NS
_SPT = _B // _NW

_HI = jax.lax.Precision.HIGHEST


def _prep_body(x_ref, eps_ref, tab_ref, idx_ref):
    e0 = eps_ref[0]
    e1 = eps_ref[1]

    def selT(sel, e):
        return lax.dot_general(sel, e, (((0,), (1,)), ((), ())),
                               preferred_element_type=jnp.float32,
                               precision=_HI)

    li = lax.broadcasted_iota(jnp.int32, (_L, _NT), 0)
    ti = lax.broadcasted_iota(jnp.int32, (_L, _NT), 1)
    s = [(li == 3 * ti + j).astype(jnp.float32) for j in range(3)]
    et = [[selT(s[j], e0), selT(s[j], e1)] for j in range(3)]
    for c in range(8):
        tc = (et[0][(c >> 2) & 1] * et[1][(c >> 1) & 1] * et[2][c & 1])
        tab_ref[pl.ds(c * _NT, _NT), :] = tc

    li2 = lax.broadcasted_iota(jnp.int32, (_L, _NP), 0)
    pi2 = lax.broadcasted_iota(jnp.int32, (_L, _NP), 1)
    pa = (li2 == 480 + 2 * pi2).astype(jnp.float32)
    pb = (li2 == 481 + 2 * pi2).astype(jnp.float32)
    ea = [selT(pa, e0), selT(pa, e1)]
    eb = [selT(pb, e0), selT(pb, e1)]
    for c in range(4):
        tab_ref[pl.ds(_TROWS + c * _NP, _NP), :] = ea[c >> 1] * eb[c & 1]

    wt = (s[0] * 4.0 + s[1] * 2.0 + s[2]) * (_NT * _M)
    wp = (pa * 2.0 + pb) * (_NP * _M)
    w = jnp.concatenate([wt, wp], axis=1)
    xf = x_ref[...].astype(jnp.float32)
    combo = lax.dot_general(xf, w, (((1,), (0,)), ((), ())),
                            preferred_element_type=jnp.float32,
                            precision=_HI)
    tcol = lax.broadcasted_iota(jnp.int32, (_B, _T), 1)
    off = tcol * _M + jnp.where(tcol >= _NT, (_TROWS - _NT) * _M, 0)
    idx_ref[...] = combo.astype(jnp.int32) + off


def _sc_body(idx_hbm, table_hbm, out_hbm, idx_v, tab_v, out_v, tab_sh):
    sid = lax.axis_index("s")
    wid = sid * _NC + lax.axis_index("c")
    base = wid * _SPT

    @pl.when(sid == 0)
    def _stage_shared():
        pltpu.sync_copy(table_hbm, tab_sh)

    pltpu.sync_copy(idx_hbm.at[pl.ds(base * _T, _SPT * _T)], idx_v)
    plsc.subcore_barrier()
    pltpu.sync_copy(tab_sh, tab_v)

    lane = lax.iota(jnp.int32, 16)
    ones = jnp.ones((_LANES,), jnp.float32)

    for g in range(_SPT // _LANES):

        def sample_body(j, outvec, g=g):
            i = g * _LANES + j

            def group_body(q, accs):
                a0, a1, a2, a3 = accs
                rbv = idx_v[pl.ds(i * _T + q * _LANES, _LANES)]
                for k in range(_LANES):
                    rb = rbv[k]
                    a0 = a0 * tab_v[pl.ds(rb, 16)]
                    a1 = a1 * tab_v[pl.ds(rb + 16, 16)]
                    a2 = a2 * tab_v[pl.ds(rb + 32, 16)]
                    a3 = a3 * tab_v[pl.ds(rb + 48, 16)]
                return (a0, a1, a2, a3)

            a0, a1, a2, a3 = lax.fori_loop(
                0, _T // _LANES, group_body, (ones, ones, ones, ones))
            tot = jnp.sum((a0 + a1) + (a2 + a3), axis=0)
            return jnp.where(lane == j, tot, outvec)

        outvec = lax.fori_loop(0, _LANES, sample_body,
                               jnp.zeros((_LANES,), jnp.float32))
        out_v[pl.ds(g * _LANES, _LANES)] = outvec

    pltpu.sync_copy(out_v, out_hbm.at[pl.ds(base, _SPT)])


@jax.jit
def _qgps(x, epsilon):
    tab, idx = pl.pallas_call(
        _prep_body,
        out_shape=(
            jax.ShapeDtypeStruct((_ROWS, _M), jnp.float32),
            jax.ShapeDtypeStruct((_B, _T), jnp.int32),
        ),
    )(x, epsilon)

    run = pl.kernel(
        _sc_body,
        out_type=jax.ShapeDtypeStruct((_B,), jnp.float32),
        mesh=plsc.VectorSubcoreMesh(core_axis_name="c", subcore_axis_name="s"),
        scratch_types=[
            pltpu.VMEM((_SPT * _T,), jnp.int32),
            pltpu.VMEM((_ROWS * _M,), jnp.float32),
            pltpu.VMEM((_SPT,), jnp.float32),
            pltpu.VMEM_SHARED((_ROWS * _M,), jnp.float32),
        ],
        compiler_params=pltpu.CompilerParams(needs_layout_passes=False),
    )
    return run(idx.reshape(-1), tab.reshape(-1))


def kernel(inputs, epsilon):
    return _qgps(inputs.astype(jnp.int32), epsilon)


# speedup vs baseline: 1.4268x; 1.0314x over previous
import functools

import jax
import jax.numpy as jnp
from jax import lax
from jax.experimental import pallas as pl
from jax.experimental.pallas import tpu as pltpu
from jax.experimental.pallas import tpu_sc as plsc

_B = 1024
_L = 512
_M = 64
_LANES = 16

_NQ = 80
_NT = 64
_T = _NQ + _NT
_QROWS = 16 * _NQ
_ROWS = _QROWS + 8 * _NT

_info = plsc.get_sparse_core_info()
_NC = _info.num_cores
_NS = _info.num_subcores
_NW = _NC * _NS
_SPT = _B // _NW

_HI = jax.lax.Precision.HIGHEST


def _prep_body(x_ref, eps_ref, tab_ref, idx_ref):
    e0 = eps_ref[0]
    e1 = eps_ref[1]

    def selT(sel, e):
        return lax.dot_general(sel, e, (((0,), (1,)), ((), ())),
                               preferred_element_type=jnp.float32,
                               precision=_HI)

    li = lax.broadcasted_iota(jnp.int32, (_L, _NQ), 0)
    qi = lax.broadcasted_iota(jnp.int32, (_L, _NQ), 1)
    s = [(li == 4 * qi + j).astype(jnp.float32) for j in range(4)]
    eq = [[selT(s[j], e0), selT(s[j], e1)] for j in range(4)]
    for c in range(16):
        tc = (eq[0][(c >> 3) & 1] * eq[1][(c >> 2) & 1]
              * eq[2][(c >> 1) & 1] * eq[3][c & 1])
        tab_ref[pl.ds(c * _NQ, _NQ), :] = tc

    li3 = lax.broadcasted_iota(jnp.int32, (_L, _NT), 0)
    ti3 = lax.broadcasted_iota(jnp.int32, (_L, _NT), 1)
    s3 = [(li3 == 4 * _NQ + 3 * ti3 + j).astype(jnp.float32)
          for j in range(3)]
    et = [[selT(s3[j], e0), selT(s3[j], e1)] for j in range(3)]
    for c in range(8):
        tc = (et[0][(c >> 2) & 1] * et[1][(c >> 1) & 1] * et[2][c & 1])
        tab_ref[pl.ds(_QROWS + c * _NT, _NT), :] = tc

    wq = (s[0] * 8.0 + s[1] * 4.0 + s[2] * 2.0 + s[3]) * (_NQ * _M)
    wt = (s3[0] * 4.0 + s3[1] * 2.0 + s3[2]) * (_NT * _M)
    w = jnp.concatenate([wq, wt], axis=1)
    xf = x_ref[...].astype(jnp.float32)
    combo = lax.dot_general(xf, w, (((1,), (0,)), ((), ())),
                            preferred_element_type=jnp.float32,
                            precision=_HI)
    tcol = lax.broadcasted_iota(jnp.int32, (_B, _T), 1)
    off = tcol * _M + jnp.where(tcol >= _NQ, (_QROWS - _NQ) * _M, 0)
    idx_ref[...] = combo.astype(jnp.int32) + off


def _sc_body(idx_hbm, table_hbm, out_hbm, idx_v, tab_v, out_v, tab_sh):
    sid = lax.axis_index("s")
    wid = sid * _NC + lax.axis_index("c")
    base = wid * _SPT

    @pl.when(sid == 0)
    def _stage_shared():
        pltpu.sync_copy(table_hbm, tab_sh)

    pltpu.sync_copy(idx_hbm.at[pl.ds(base * _T, _SPT * _T)], idx_v)
    plsc.subcore_barrier()
    pltpu.sync_copy(tab_sh, tab_v)

    lane = lax.iota(jnp.int32, 16)
    ones = jnp.ones((_LANES,), jnp.float32)

    for g in range(_SPT // _LANES):

        def sample_body(j, outvec, g=g):
            i = g * _LANES + j

            def group_body(q, accs):
                a0, a1, a2, a3 = accs
                rbv = idx_v[pl.ds(i * _T + q * _LANES, _LANES)]
                for k in range(_LANES):
                    rb = rbv[k]
                    a0 = a0 * tab_v[pl.ds(rb, 16)]
                    a1 = a1 * tab_v[pl.ds(rb + 16, 16)]
                    a2 = a2 * tab_v[pl.ds(rb + 32, 16)]
                    a3 = a3 * tab_v[pl.ds(rb + 48, 16)]
                return (a0, a1, a2, a3)

            a0, a1, a2, a3 = lax.fori_loop(
                0, _T // _LANES, group_body, (ones, ones, ones, ones))
            tot = jnp.sum((a0 + a1) + (a2 + a3), axis=0)
            return jnp.where(lane == j, tot, outvec)

        outvec = lax.fori_loop(0, _LANES, sample_body,
                               jnp.zeros((_LANES,), jnp.float32))
        out_v[pl.ds(g * _LANES, _LANES)] = outvec

    pltpu.sync_copy(out_v, out_hbm.at[pl.ds(base, _SPT)])


@jax.jit
def _qgps(x, epsilon):
    tab, idx = pl.pallas_call(
        _prep_body,
        out_shape=(
            jax.ShapeDtypeStruct((_ROWS, _M), jnp.float32),
            jax.ShapeDtypeStruct((_B, _T), jnp.int32),
        ),
    )(x, epsilon)

    run = pl.kernel(
        _sc_body,
        out_type=jax.ShapeDtypeStruct((_B,), jnp.float32),
        mesh=plsc.VectorSubcoreMesh(core_axis_name="c", subcore_axis_name="s"),
        scratch_types=[
            pltpu.VMEM((_SPT * _T,), jnp.int32),
            pltpu.VMEM((_ROWS * _M,), jnp.float32),
            pltpu.VMEM((_SPT,), jnp.float32),
            pltpu.VMEM_SHARED((_ROWS * _M,), jnp.float32),
        ],
        compiler_params=pltpu.CompilerParams(needs_layout_passes=False),
    )
    return run(idx.reshape(-1), tab.reshape(-1))


def kernel(inputs, epsilon):
    return _qgps(inputs.astype(jnp.int32), epsilon)
